# baseline (speedup 1.0000x reference)
import functools

import jax
import jax.numpy as jnp
from jax import lax
from jax.experimental import pallas as pl
from jax.experimental.pallas import tpu as pltpu
from jax.experimental.pallas import tpu_sc as plsc

_CH = 128
_NCH = 4
_L = 16


def _extract(staged, selb, outb, ch, D):
    for g in range(_CH // _L):
        sv = selb[pl.ds(ch * _CH + g * _L, _L)] * D
        for l in range(_L):
            r = g * _L + l
            off = sv[l]
            for k in range(D // _L):
                outb[r, pl.ds(k * _L, _L)] = staged[r, pl.ds(off + k * _L, _L)]


def _body(b_per_w, D, tile_hbm, sel_hbm, tbl2, out_hbm,
          idxb, selb, st0, st1, outb, sem0, sem1):
    wid = lax.axis_index("s") * 2 + lax.axis_index("c")
    base = wid * b_per_w

    pltpu.sync_copy(tile_hbm.at[wid], idxb)
    pltpu.sync_copy(sel_hbm.at[pl.ds(base, b_per_w)], selb)

    def fire(ch, buf, sem):
        pltpu.async_copy(tbl2.at[idxb.at[ch]], buf, sem)

    def wait_chunk(buf, sem):
        pltpu.make_async_copy(tbl2.at[pl.ds(0, _CH)], buf, sem).wait()

    bufs = ((st0, sem0), (st1, sem1))
    fire(0, st0, sem0)
    fire(1, st1, sem1)

    def step(p, carry):
        for q in range(2):
            ch = 2 * p + q
            buf, sem = bufs[q]
            wait_chunk(buf, sem)
            _extract(buf, selb, outb, ch, D)
            pltpu.sync_copy(outb, out_hbm.at[pl.ds(base + ch * _CH, _CH)])
            fire(jnp.remainder(ch + 2, _NCH), buf, sem)
        return carry

    lax.fori_loop(0, _NCH // 2, step, 0)
    wait_chunk(st0, sem0)
    wait_chunk(st1, sem1)


def _make_gather(B, V, D, b_per_w, nw):
    mesh = plsc.VectorSubcoreMesh(core_axis_name="c", subcore_axis_name="s")
    return pl.kernel(
        functools.partial(_body, b_per_w, D),
        out_type=jax.ShapeDtypeStruct((B, D), jnp.float32),
        mesh=mesh,
        scratch_types=[
            pltpu.VMEM((_NCH, _CH), jnp.int32),
            pltpu.VMEM((b_per_w,), jnp.int32),
            pltpu.VMEM((_CH, 2 * D), jnp.float32),
            pltpu.VMEM((_CH, 2 * D), jnp.float32),
            pltpu.VMEM((_CH, D), jnp.float32),
            pltpu.SemaphoreType.DMA,
            pltpu.SemaphoreType.DMA,
        ],
        compiler_params=pltpu.CompilerParams(needs_layout_passes=False),
    )


def kernel(user_ids, item_ids, user_table, item_table):
    B = user_ids.shape[0]
    V, D = user_table.shape
    info = plsc.get_sparse_core_info()
    nw = info.num_cores * info.num_subcores
    b_per_w = B // nw
    assert b_per_w == _CH * _NCH

    uid = user_ids.astype(jnp.int32)
    iid = item_ids.astype(jnp.int32)
    run = _make_gather(B, V, D, b_per_w, nw)
    user_emb = run((uid >> 1).reshape(nw, _NCH, _CH), uid & 1,
                   user_table.reshape(V // 2, 2 * D))
    item_emb = run((iid >> 1).reshape(nw, _NCH, _CH), iid & 1,
                   item_table.reshape(V // 2, 2 * D))
    return (user_emb, user_emb, item_emb, item_emb)


# speedup vs baseline: 2.0812x; 2.0812x over previous
import functools

import jax
import jax.numpy as jnp
from jax import lax
from jax.experimental import pallas as pl
from jax.experimental.pallas import tpu as pltpu
from jax.experimental.pallas import tpu_sc as plsc

_CH = 32
_NCH = 16
_L = 16
_QC = 4


def _fire(tbl3, idxb, ch, buf, sem):
    for g in range(_CH // _L):
        v = idxb[pl.ds(ch * _CH + g * _L, _L)]
        for l in range(_L):
            pltpu.async_copy(tbl3.at[v[l]], buf.at[g * _L + l], sem)


def _wait_chunk(tbl3, buf, sem):
    pltpu.make_async_copy(tbl3.at[pl.ds(0, _CH)], buf, sem).wait()


def _extract(staged, selb, outbT, ch, q, D):
    for g in range(_CH // _L):
        rows = lax.iota(jnp.int32, _L) + g * _L
        sel_vec = selb[pl.ds(ch * _CH + g * _L, _L)]
        for c in range(D):
            cvec = jnp.full((_L,), c, jnp.int32)
            vals = plsc.load_gather(staged, [rows, sel_vec, cvec])
            outbT[c, pl.ds(q * _CH + g * _L, _L)] = vals


def _body(b_per_w, D, tile_hbm, sel_hbm, tbl3, outT,
          idxb, selb, st0, st1, outbT, sem0, sem1):
    wid = lax.axis_index("s") * 2 + lax.axis_index("c")
    base = wid * b_per_w

    pltpu.sync_copy(tile_hbm.at[pl.ds(base, b_per_w)], idxb)
    pltpu.sync_copy(sel_hbm.at[pl.ds(base, b_per_w)], selb)

    bufs = ((st0, sem0), (st1, sem1))
    _fire(tbl3, idxb, 0, st0, sem0)
    _fire(tbl3, idxb, 1, st1, sem1)

    def step(p, carry):
        for q in range(_QC):
            ch = _QC * p + q
            buf, sem = bufs[q % 2]
            _wait_chunk(tbl3, buf, sem)
            _extract(buf, selb, outbT, ch, q, D)
            _fire(tbl3, idxb, jnp.remainder(ch + 2, _NCH), buf, sem)
        pltpu.sync_copy(outbT,
                        outT.at[:, pl.ds(base + p * _QC * _CH, _QC * _CH)])
        return carry

    lax.fori_loop(0, _NCH // _QC, step, 0)
    _wait_chunk(tbl3, st0, sem0)
    _wait_chunk(tbl3, st1, sem1)


def _make_gather(B, V, D, b_per_w):
    mesh = plsc.VectorSubcoreMesh(core_axis_name="c", subcore_axis_name="s")
    return pl.kernel(
        functools.partial(_body, b_per_w, D),
        out_type=jax.ShapeDtypeStruct((D, B), jnp.float32),
        mesh=mesh,
        scratch_types=[
            pltpu.VMEM((b_per_w,), jnp.int32),
            pltpu.VMEM((b_per_w,), jnp.int32),
            pltpu.VMEM((_CH, 8, D), jnp.float32),
            pltpu.VMEM((_CH, 8, D), jnp.float32),
            pltpu.VMEM((D, _QC * _CH), jnp.float32),
            pltpu.SemaphoreType.DMA,
            pltpu.SemaphoreType.DMA,
        ],
        compiler_params=pltpu.CompilerParams(needs_layout_passes=False),
    )


def kernel(user_ids, item_ids, user_table, item_table):
    B = user_ids.shape[0]
    V, D = user_table.shape
    info = plsc.get_sparse_core_info()
    nw = info.num_cores * info.num_subcores
    b_per_w = B // nw
    assert b_per_w == _CH * _NCH

    uid = user_ids.astype(jnp.int32)
    iid = item_ids.astype(jnp.int32)
    run = _make_gather(B, V, D, b_per_w)
    u_t = run(uid >> 3, uid & 7, user_table.reshape(V // 8, 8, D))
    i_t = run(iid >> 3, iid & 7, item_table.reshape(V // 8, 8, D))
    user_emb = u_t.T
    item_emb = i_t.T
    return (user_emb, user_emb, item_emb, item_emb)
